# baseline (speedup 1.0000x reference)
import functools

import jax
import jax.numpy as jnp
from jax import lax
from jax.experimental import pallas as pl
from jax.experimental.pallas import tpu as pltpu
from jax.experimental.pallas import tpu_sc as plsc

HEADS = 8
CDIM = 16
LANES = HEADS * CDIM
SCALE_LIN = 32.0 / 8.0
SCALE_ATT = 32.0 / 4.0
NEG_SLOPE = 0.2

NC = 2
NS = 16
CH = 96
ZR = 12


def _dot_t(a, b):
    return lax.dot_general(a, b, (((1,), (1,)), ((), ())),
                           preferred_element_type=jnp.float32,
                           precision=lax.Precision.HIGHEST)


def _node_transform(xp_like, Wb, A, B, asrc_f, adst_f, Asrc, Bsrc, Adst, Bdst):
    xp = _dot_t(xp_like, Wb)
    u = _dot_t(xp_like, A)
    xp = xp + _dot_t(u, B) * SCALE_LIN
    lor_d = lax.dot_general(Bsrc, Asrc, (((1,), (0,)), ((), ())),
                            preferred_element_type=jnp.float32,
                            precision=lax.Precision.HIGHEST) * SCALE_ATT
    lor_s = lax.dot_general(Bdst, Adst, (((1,), (0,)), ((), ())),
                            preferred_element_type=jnp.float32,
                            precision=lax.Precision.HIGHEST) * SCALE_ATT
    aeff_d = asrc_f + jnp.concatenate([lor_d] * HEADS, axis=1)
    aeff_s = adst_f + jnp.concatenate([lor_s] * HEADS, axis=1)
    rr = lax.broadcasted_iota(jnp.int32, (LANES, CDIM), 0)
    cc = lax.broadcasted_iota(jnp.int32, (LANES, CDIM), 1)
    ssum = jnp.where((rr // CDIM) == cc, 1.0, 0.0).astype(jnp.float32)
    sA = jnp.dot(xp * aeff_d, ssum, preferred_element_type=jnp.float32,
                 precision=lax.Precision.HIGHEST)
    sB = jnp.dot(xp * aeff_s, ssum, preferred_element_type=jnp.float32,
                 precision=lax.Precision.HIGHEST)
    return xp, sA, sB


def _tc1_body(x_ref, Wb_ref, A_ref, B_ref, asrc_ref, adst_ref,
              Asrc_ref, Bsrc_ref, Adst_ref, Bdst_ref,
              xp_ref, sA_ref, sB_ref):
    xp, sA, sB = _node_transform(
        x_ref[...], Wb_ref[...], A_ref[...], B_ref[...],
        asrc_ref[...], adst_ref[...], Asrc_ref[...], Bsrc_ref[...],
        Adst_ref[...], Bdst_ref[...])
    xp_ref[...] = xp
    sA_ref[...] = sA
    sB_ref[...] = sB


def _expand_mat():
    hh = lax.broadcasted_iota(jnp.int32, (CDIM, LANES), 0)
    rr = lax.broadcasted_iota(jnp.int32, (CDIM, LANES), 1)
    return jnp.where(hh == (rr // CDIM), 1.0, 0.0).astype(jnp.float32)


def _tc2_body(msg_ref, den_ref, x_ref, bias1_ref, Wb_ref, A_ref, B_ref,
              asrc_ref, adst_ref, Asrc_ref, Bsrc_ref, Adst_ref, Bdst_ref,
              xp_ref, sA_ref, sB_ref):
    msg = msg_ref[0] + msg_ref[1]
    den = den_ref[0] + den_ref[1]
    rden = 1.0 / (den + 1e-16)
    out1 = msg * jnp.dot(rden, _expand_mat(),
                         preferred_element_type=jnp.float32,
                         precision=lax.Precision.HIGHEST) + bias1_ref[...]
    h = jnp.where(out1 > 0, out1, jnp.exp(out1) - 1.0) + x_ref[...]
    xp, sA, sB = _node_transform(
        h, Wb_ref[...], A_ref[...], B_ref[...],
        asrc_ref[...], adst_ref[...], Asrc_ref[...], Bsrc_ref[...],
        Adst_ref[...], Bdst_ref[...])
    xp_ref[...] = xp
    sA_ref[...] = sA
    sB_ref[...] = sB


def _tc3_body(msg_ref, den_ref, bias2_ref, out_ref):
    msg = msg_ref[0] + msg_ref[1]
    den = den_ref[0] + den_ref[1]
    rden = 1.0 / (den + 1e-16)
    q = msg * jnp.dot(rden, _expand_mat(), preferred_element_type=jnp.float32,
                      precision=lax.Precision.HIGHEST)
    rr = lax.broadcasted_iota(jnp.int32, (LANES, CDIM), 0)
    cc = lax.broadcasted_iota(jnp.int32, (LANES, CDIM), 1)
    meanm = jnp.where((rr % CDIM) == cc, 1.0 / HEADS, 0.0).astype(jnp.float32)
    out_ref[...] = jnp.dot(q, meanm, preferred_element_type=jnp.float32,
                           precision=lax.Precision.HIGHEST) + bias2_ref[...]


def _full_spec(shape):
    nd = len(shape)
    return pl.BlockSpec(shape, lambda i: (0,) * nd)


def _tc_node1(x_pad, Wb, A, B, asrc_f, adst_f, Asrc, Bsrc, Adst, Bdst,
              npad, blk):
    grid = (npad // blk,)
    return pl.pallas_call(
        _tc1_body,
        grid=grid,
        in_specs=[
            pl.BlockSpec((blk, LANES), lambda i: (i, 0)),
            _full_spec(Wb.shape), _full_spec(A.shape), _full_spec(B.shape),
            _full_spec(asrc_f.shape), _full_spec(adst_f.shape),
            _full_spec(Asrc.shape), _full_spec(Bsrc.shape),
            _full_spec(Adst.shape), _full_spec(Bdst.shape),
        ],
        out_specs=[
            pl.BlockSpec((blk, LANES), lambda i: (i, 0)),
            pl.BlockSpec((blk, CDIM), lambda i: (i, 0)),
            pl.BlockSpec((blk, CDIM), lambda i: (i, 0)),
        ],
        out_shape=[
            jax.ShapeDtypeStruct((npad, LANES), jnp.float32),
            jax.ShapeDtypeStruct((npad, CDIM), jnp.float32),
            jax.ShapeDtypeStruct((npad, CDIM), jnp.float32),
        ],
    )(x_pad, Wb, A, B, asrc_f, adst_f, Asrc, Bsrc, Adst, Bdst)


def _tc_mid(msg_p, den_p, x_pad, bias1_f, Wb, A, B, asrc_f, adst_f,
            Asrc, Bsrc, Adst, Bdst, npad, blk):
    grid = (npad // blk,)
    return pl.pallas_call(
        _tc2_body,
        grid=grid,
        in_specs=[
            pl.BlockSpec((NC, blk, LANES), lambda i: (0, i, 0)),
            pl.BlockSpec((NC, blk, CDIM), lambda i: (0, i, 0)),
            pl.BlockSpec((blk, LANES), lambda i: (i, 0)),
            _full_spec(bias1_f.shape),
            _full_spec(Wb.shape), _full_spec(A.shape), _full_spec(B.shape),
            _full_spec(asrc_f.shape), _full_spec(adst_f.shape),
            _full_spec(Asrc.shape), _full_spec(Bsrc.shape),
            _full_spec(Adst.shape), _full_spec(Bdst.shape),
        ],
        out_specs=[
            pl.BlockSpec((blk, LANES), lambda i: (i, 0)),
            pl.BlockSpec((blk, CDIM), lambda i: (i, 0)),
            pl.BlockSpec((blk, CDIM), lambda i: (i, 0)),
        ],
        out_shape=[
            jax.ShapeDtypeStruct((npad, LANES), jnp.float32),
            jax.ShapeDtypeStruct((npad, CDIM), jnp.float32),
            jax.ShapeDtypeStruct((npad, CDIM), jnp.float32),
        ],
    )(msg_p, den_p, x_pad, bias1_f, Wb, A, B, asrc_f, adst_f,
      Asrc, Bsrc, Adst, Bdst)


def _tc_final(msg_p, den_p, bias2_f, npad, blk):
    grid = (npad // blk,)
    return pl.pallas_call(
        _tc3_body,
        grid=grid,
        in_specs=[
            pl.BlockSpec((NC, blk, LANES), lambda i: (0, i, 0)),
            pl.BlockSpec((NC, blk, CDIM), lambda i: (0, i, 0)),
            _full_spec(bias2_f.shape),
        ],
        out_specs=pl.BlockSpec((blk, CDIM), lambda i: (i, 0)),
        out_shape=jax.ShapeDtypeStruct((npad, CDIM), jnp.float32),
    )(msg_p, den_p, bias2_f)


def _sc_edge(xp, sA, sB, packed2d, npad, k_chunks):
    rows_per_sub = npad // NS
    mesh = plsc.VectorSubcoreMesh(core_axis_name="c", subcore_axis_name="s")
    out_type = [
        jax.ShapeDtypeStruct((NC, npad, LANES), jnp.float32),
        jax.ShapeDtypeStruct((NC, npad, CDIM), jnp.float32),
    ]
    scratch = [
        pltpu.VMEM((12, CH), jnp.int32),
        pltpu.VMEM((2, CH), jnp.int32),
        pltpu.VMEM((2, CH), jnp.int32),
        pltpu.VMEM((2 * CH, LANES), jnp.float32),
        pltpu.VMEM((2 * CH, CDIM), jnp.float32),
        pltpu.VMEM((2 * CH, CDIM), jnp.float32),
        pltpu.VMEM((CH, CDIM), jnp.float32),
        pltpu.VMEM((ZR, LANES), jnp.float32),
        pltpu.VMEM((ZR, CDIM), jnp.float32),
        pltpu.VMEM_SHARED((npad, LANES), jnp.float32),
        pltpu.VMEM_SHARED((npad, CDIM), jnp.float32),
        pltpu.SemaphoreType.DMA,
        pltpu.SemaphoreType.DMA,
    ]

    @functools.partial(pl.kernel, out_type=out_type, mesh=mesh,
                       scratch_types=scratch,
                       compiler_params=pltpu.CompilerParams(
                           use_tc_tiling_on_sc=False))
    def body(xp_hbm, sA_hbm, sB_hbm, pk_hbm, msg_out, den_out,
             pk_v, src_v, dst_v, xp_rows, a_rows, b_rows, ex_rows,
             zbuf, zbuf16, msg_acc, den_acc, sem, sem2):
        cid = lax.axis_index("c")
        sid = lax.axis_index("s")
        wid = cid * NS + sid
        base_row = sid * rows_per_sub

        def zfill(i, carry):
            for j in range(HEADS):
                zbuf[i, pl.ds(j * 16, 16)] = jnp.zeros((16,), jnp.float32)
            zbuf16[i, :] = jnp.zeros((16,), jnp.float32)
            return carry
        lax.fori_loop(0, ZR, zfill, 0)

        def zcopy(t, carry):
            r = base_row + t * ZR
            pltpu.async_copy(zbuf, msg_acc.at[pl.ds(r, ZR)], sem)
            pltpu.async_copy(zbuf16, den_acc.at[pl.ds(r, ZR)], sem)
            return carry
        lax.fori_loop(0, rows_per_sub // ZR, zcopy, 0)

        def zdrain(t, carry):
            r = base_row + t * ZR
            pltpu.make_async_copy(zbuf, msg_acc.at[pl.ds(r, ZR)], sem).wait()
            pltpu.make_async_copy(zbuf16, den_acc.at[pl.ds(r, ZR)],
                                  sem).wait()
            return carry
        lax.fori_loop(0, rows_per_sub // ZR, zdrain, 0)
        plsc.subcore_barrier()

        row0 = wid * k_chunks

        def ring(k, carry):
            s = k % 2

            @pl.when((k >= 2) & (k <= k_chunks + 1))
            def _drain_scatter():
                pltpu.make_async_copy(
                    ex_rows, den_acc.at[dst_v.at[s]], sem2).wait()
                pltpu.make_async_copy(
                    xp_rows.at[pl.ds(s * CH, CH)],
                    msg_acc.at[dst_v.at[s]], sem2).wait()

            @pl.when(k < k_chunks)
            def _fire_gather():
                @pl.when(k % 12 == 0)
                def _refill_idx():
                    pltpu.sync_copy(pk_hbm.at[pl.ds(row0 + k, 12)], pk_v)
                for t in range(CH // 16):
                    v = pk_v[k % 12, pl.ds(t * 16, 16)]
                    src_v[s, pl.ds(t * 16, 16)] = v & 16383
                    dst_v[s, pl.ds(t * 16, 16)] = v >> 14
                pltpu.async_copy(xp_hbm.at[src_v.at[s]],
                                 xp_rows.at[pl.ds(s * CH, CH)], sem)
                pltpu.async_copy(sA_hbm.at[dst_v.at[s]],
                                 a_rows.at[pl.ds(s * CH, CH)], sem)
                pltpu.async_copy(sB_hbm.at[src_v.at[s]],
                                 b_rows.at[pl.ds(s * CH, CH)], sem)

            @pl.when((k >= 1) & (k <= k_chunks))
            def _compute():
                p = 1 - s
                pltpu.make_async_copy(xp_hbm.at[src_v.at[p]],
                                      xp_rows.at[pl.ds(p * CH, CH)],
                                      sem).wait()
                pltpu.make_async_copy(sA_hbm.at[dst_v.at[p]],
                                      a_rows.at[pl.ds(p * CH, CH)],
                                      sem).wait()
                pltpu.make_async_copy(sB_hbm.at[src_v.at[p]],
                                      b_rows.at[pl.ds(p * CH, CH)],
                                      sem).wait()
                base = p * CH

                @plsc.parallel_loop(0, CH, 1, unroll=8)
                def _edges(e):
                    i = base + e
                    v = a_rows[i, :] + b_rows[i, :]
                    v = jnp.where(v >= 0, v, NEG_SLOPE * v)
                    v = jnp.exp(v)
                    ex_rows[e, :] = v
                    for h in range(HEADS):
                        bro = v.at[jnp.full((16,), h, jnp.int32)].get(
                            mode="promise_in_bounds")
                        xp_rows[i, pl.ds(h * 16, 16)] = (
                            xp_rows[i, pl.ds(h * 16, 16)] * bro)

                pltpu.async_copy(ex_rows, den_acc.at[dst_v.at[p]], sem2,
                                 add=True)
                pltpu.async_copy(xp_rows.at[pl.ds(p * CH, CH)],
                                 msg_acc.at[dst_v.at[p]], sem2, add=True)
            return carry
        lax.fori_loop(0, k_chunks + 2, ring, 0)
        plsc.subcore_barrier()

        def wout(t, carry):
            r = base_row + t * ZR
            pltpu.async_copy(msg_acc.at[pl.ds(r, ZR)],
                             msg_out.at[cid, pl.ds(r, ZR)], sem)
            pltpu.async_copy(den_acc.at[pl.ds(r, ZR)],
                             den_out.at[cid, pl.ds(r, ZR)], sem)
            return carry
        lax.fori_loop(0, rows_per_sub // ZR, wout, 0)

        def wdrain(t, carry):
            r = base_row + t * ZR
            pltpu.make_async_copy(msg_acc.at[pl.ds(r, ZR)],
                                  msg_out.at[cid, pl.ds(r, ZR)], sem).wait()
            pltpu.make_async_copy(den_acc.at[pl.ds(r, ZR)],
                                  den_out.at[cid, pl.ds(r, ZR)], sem).wait()
            return carry
        lax.fori_loop(0, rows_per_sub // ZR, wdrain, 0)

    return body(xp, sA, sB, packed2d)


def kernel(x, edge_index, Wb1, A1, B1, att_src1, att_dst1, Asrc1, Bsrc1,
           Adst1, Bdst1, bias1, Wb2, A2, B2, att_src2, att_dst2, Asrc2,
           Bsrc2, Adst2, Bdst2, bias2):
    n = x.shape[0]
    e_raw = edge_index.shape[1]
    gran = NS * ZR
    npad = ((n + 1 + gran - 1) // gran) * gran
    blk = npad // 8
    assert npad <= 16384

    ep = e_raw + n
    per_round = NC * NS * CH
    k_chunks = (ep + per_round - 1) // per_round
    k_chunks = ((k_chunks + 11) // 12) * 12
    epad = k_chunks * per_round
    pad_e = epad - ep

    loops = jnp.arange(n, dtype=jnp.int32)
    src = jnp.concatenate([edge_index[0].astype(jnp.int32), loops,
                           jnp.zeros((pad_e,), jnp.int32)])
    dst = jnp.concatenate([edge_index[1].astype(jnp.int32), loops,
                           jnp.full((pad_e,), n, jnp.int32)])
    packed2d = (dst * 16384 + src).reshape(-1, CH)

    x_pad = jnp.pad(x, ((0, npad - n), (0, 0)))
    asrc1_f = att_src1.reshape(1, LANES)
    adst1_f = att_dst1.reshape(1, LANES)
    asrc2_f = att_src2.reshape(1, LANES)
    adst2_f = att_dst2.reshape(1, LANES)
    bias1_f = bias1.reshape(1, LANES)
    bias2_f = bias2.reshape(1, CDIM)

    xp1, sA1, sB1 = _tc_node1(x_pad, Wb1, A1, B1, asrc1_f, adst1_f,
                              Asrc1, Bsrc1, Adst1, Bdst1, npad, blk)
    msg1, den1 = _sc_edge(xp1, sA1, sB1, packed2d, npad, k_chunks)
    xp2, sA2, sB2 = _tc_mid(msg1, den1, x_pad, bias1_f, Wb2, A2, B2,
                            asrc2_f, adst2_f, Asrc2, Bsrc2, Adst2, Bdst2,
                            npad, blk)
    msg2, den2 = _sc_edge(xp2, sA2, sB2, packed2d, npad, k_chunks)
    out = _tc_final(msg2, den2, bias2_f, npad, blk)
    return out[:n]


# speedup vs baseline: 1.2145x; 1.2145x over previous
import functools

import jax
import jax.numpy as jnp
from jax import lax
from jax.experimental import pallas as pl
from jax.experimental.pallas import tpu as pltpu
from jax.experimental.pallas import tpu_sc as plsc

HEADS = 8
CDIM = 16
LANES = HEADS * CDIM
SCALE_LIN = 32.0 / 8.0
SCALE_ATT = 32.0 / 4.0
NEG_SLOPE = 0.2

NC = 2
NS = 16
CH = 96
ZR = 12


def _dot_t(a, b):
    return lax.dot_general(a, b, (((1,), (1,)), ((), ())),
                           preferred_element_type=jnp.float32,
                           precision=lax.Precision.HIGHEST)


def _node_transform(xp_like, Wb, A, B, asrc_f, adst_f, Asrc, Bsrc, Adst, Bdst):
    xp = _dot_t(xp_like, Wb)
    u = _dot_t(xp_like, A)
    xp = xp + _dot_t(u, B) * SCALE_LIN
    lor_d = lax.dot_general(Bsrc, Asrc, (((1,), (0,)), ((), ())),
                            preferred_element_type=jnp.float32,
                            precision=lax.Precision.HIGHEST) * SCALE_ATT
    lor_s = lax.dot_general(Bdst, Adst, (((1,), (0,)), ((), ())),
                            preferred_element_type=jnp.float32,
                            precision=lax.Precision.HIGHEST) * SCALE_ATT
    aeff_d = asrc_f + jnp.concatenate([lor_d] * HEADS, axis=1)
    aeff_s = adst_f + jnp.concatenate([lor_s] * HEADS, axis=1)
    rr = lax.broadcasted_iota(jnp.int32, (LANES, CDIM), 0)
    cc = lax.broadcasted_iota(jnp.int32, (LANES, CDIM), 1)
    ssum = jnp.where((rr // CDIM) == cc, 1.0, 0.0).astype(jnp.float32)
    sA = jnp.dot(xp * aeff_d, ssum, preferred_element_type=jnp.float32,
                 precision=lax.Precision.HIGHEST)
    sB = jnp.dot(xp * aeff_s, ssum, preferred_element_type=jnp.float32,
                 precision=lax.Precision.HIGHEST)
    return xp, sA, sB


def _tc1_body(x_ref, Wb_ref, A_ref, B_ref, asrc_ref, adst_ref,
              Asrc_ref, Bsrc_ref, Adst_ref, Bdst_ref,
              xp_ref, sA_ref, sB_ref):
    xp, sA, sB = _node_transform(
        x_ref[...], Wb_ref[...], A_ref[...], B_ref[...],
        asrc_ref[...], adst_ref[...], Asrc_ref[...], Bsrc_ref[...],
        Adst_ref[...], Bdst_ref[...])
    xp_ref[...] = xp
    sA_ref[...] = sA
    sB_ref[...] = sB


def _expand_mat():
    hh = lax.broadcasted_iota(jnp.int32, (CDIM, LANES), 0)
    rr = lax.broadcasted_iota(jnp.int32, (CDIM, LANES), 1)
    return jnp.where(hh == (rr // CDIM), 1.0, 0.0).astype(jnp.float32)


def _tc2_body(msg_ref, den_ref, x_ref, bias1_ref, Wb_ref, A_ref, B_ref,
              asrc_ref, adst_ref, Asrc_ref, Bsrc_ref, Adst_ref, Bdst_ref,
              xp_ref, sA_ref, sB_ref):
    msg = msg_ref[0] + msg_ref[1]
    den = den_ref[0] + den_ref[1]
    rden = 1.0 / (den + 1e-16)
    out1 = msg * jnp.dot(rden, _expand_mat(),
                         preferred_element_type=jnp.float32,
                         precision=lax.Precision.HIGHEST) + bias1_ref[...]
    h = jnp.where(out1 > 0, out1, jnp.exp(out1) - 1.0) + x_ref[...]
    xp, sA, sB = _node_transform(
        h, Wb_ref[...], A_ref[...], B_ref[...],
        asrc_ref[...], adst_ref[...], Asrc_ref[...], Bsrc_ref[...],
        Adst_ref[...], Bdst_ref[...])
    xp_ref[...] = xp
    sA_ref[...] = sA
    sB_ref[...] = sB


def _tc3_body(msg_ref, den_ref, bias2_ref, out_ref):
    msg = msg_ref[0] + msg_ref[1]
    den = den_ref[0] + den_ref[1]
    rden = 1.0 / (den + 1e-16)
    q = msg * jnp.dot(rden, _expand_mat(), preferred_element_type=jnp.float32,
                      precision=lax.Precision.HIGHEST)
    rr = lax.broadcasted_iota(jnp.int32, (LANES, CDIM), 0)
    cc = lax.broadcasted_iota(jnp.int32, (LANES, CDIM), 1)
    meanm = jnp.where((rr % CDIM) == cc, 1.0 / HEADS, 0.0).astype(jnp.float32)
    out_ref[...] = jnp.dot(q, meanm, preferred_element_type=jnp.float32,
                           precision=lax.Precision.HIGHEST) + bias2_ref[...]


def _full_spec(shape):
    nd = len(shape)
    return pl.BlockSpec(shape, lambda i: (0,) * nd)


def _tc_node1(x_pad, Wb, A, B, asrc_f, adst_f, Asrc, Bsrc, Adst, Bdst,
              npad, blk):
    grid = (npad // blk,)
    return pl.pallas_call(
        _tc1_body,
        grid=grid,
        in_specs=[
            pl.BlockSpec((blk, LANES), lambda i: (i, 0)),
            _full_spec(Wb.shape), _full_spec(A.shape), _full_spec(B.shape),
            _full_spec(asrc_f.shape), _full_spec(adst_f.shape),
            _full_spec(Asrc.shape), _full_spec(Bsrc.shape),
            _full_spec(Adst.shape), _full_spec(Bdst.shape),
        ],
        out_specs=[
            pl.BlockSpec((blk, LANES), lambda i: (i, 0)),
            pl.BlockSpec((blk, CDIM), lambda i: (i, 0)),
            pl.BlockSpec((blk, CDIM), lambda i: (i, 0)),
        ],
        out_shape=[
            jax.ShapeDtypeStruct((npad, LANES), jnp.float32),
            jax.ShapeDtypeStruct((npad, CDIM), jnp.float32),
            jax.ShapeDtypeStruct((npad, CDIM), jnp.float32),
        ],
    )(x_pad, Wb, A, B, asrc_f, adst_f, Asrc, Bsrc, Adst, Bdst)


def _tc_mid(msg_p, den_p, x_pad, bias1_f, Wb, A, B, asrc_f, adst_f,
            Asrc, Bsrc, Adst, Bdst, npad, blk):
    grid = (npad // blk,)
    return pl.pallas_call(
        _tc2_body,
        grid=grid,
        in_specs=[
            pl.BlockSpec((NC, blk, LANES), lambda i: (0, i, 0)),
            pl.BlockSpec((NC, blk, CDIM), lambda i: (0, i, 0)),
            pl.BlockSpec((blk, LANES), lambda i: (i, 0)),
            _full_spec(bias1_f.shape),
            _full_spec(Wb.shape), _full_spec(A.shape), _full_spec(B.shape),
            _full_spec(asrc_f.shape), _full_spec(adst_f.shape),
            _full_spec(Asrc.shape), _full_spec(Bsrc.shape),
            _full_spec(Adst.shape), _full_spec(Bdst.shape),
        ],
        out_specs=[
            pl.BlockSpec((blk, LANES), lambda i: (i, 0)),
            pl.BlockSpec((blk, CDIM), lambda i: (i, 0)),
            pl.BlockSpec((blk, CDIM), lambda i: (i, 0)),
        ],
        out_shape=[
            jax.ShapeDtypeStruct((npad, LANES), jnp.float32),
            jax.ShapeDtypeStruct((npad, CDIM), jnp.float32),
            jax.ShapeDtypeStruct((npad, CDIM), jnp.float32),
        ],
    )(msg_p, den_p, x_pad, bias1_f, Wb, A, B, asrc_f, adst_f,
      Asrc, Bsrc, Adst, Bdst)


def _tc_final(msg_p, den_p, bias2_f, npad, blk):
    grid = (npad // blk,)
    return pl.pallas_call(
        _tc3_body,
        grid=grid,
        in_specs=[
            pl.BlockSpec((NC, blk, LANES), lambda i: (0, i, 0)),
            pl.BlockSpec((NC, blk, CDIM), lambda i: (0, i, 0)),
            _full_spec(bias2_f.shape),
        ],
        out_specs=pl.BlockSpec((blk, CDIM), lambda i: (i, 0)),
        out_shape=jax.ShapeDtypeStruct((npad, CDIM), jnp.float32),
    )(msg_p, den_p, bias2_f)


def _sc_edge(xp, sA, sB, packed2d, npad, k_chunks):
    rows_per_sub = npad // NS
    mesh = plsc.VectorSubcoreMesh(core_axis_name="c", subcore_axis_name="s")
    out_type = [
        jax.ShapeDtypeStruct((NC, npad, LANES), jnp.float32),
        jax.ShapeDtypeStruct((NC, npad, CDIM), jnp.float32),
    ]
    scratch = [
        pltpu.VMEM((12, CH), jnp.int32),
        pltpu.VMEM((2, CH), jnp.int32),
        pltpu.VMEM((2, CH), jnp.int32),
        pltpu.VMEM((2 * CH, LANES), jnp.float32),
        pltpu.VMEM((2 * CH, CDIM), jnp.float32),
        pltpu.VMEM((2 * CH, CDIM), jnp.float32),
        pltpu.VMEM((CH, CDIM), jnp.float32),
        pltpu.VMEM((ZR, LANES), jnp.float32),
        pltpu.VMEM((ZR, CDIM), jnp.float32),
        pltpu.VMEM_SHARED((npad, LANES), jnp.float32),
        pltpu.VMEM_SHARED((npad, CDIM), jnp.float32),
        pltpu.SemaphoreType.DMA,
        pltpu.SemaphoreType.DMA,
    ]

    @functools.partial(pl.kernel, out_type=out_type, mesh=mesh,
                       scratch_types=scratch,
                       compiler_params=pltpu.CompilerParams(
                           use_tc_tiling_on_sc=False))
    def body(xp_hbm, sA_hbm, sB_hbm, pk_hbm, msg_out, den_out,
             pk_v, src_v, dst_v, xp_rows, a_rows, b_rows, ex_rows,
             zbuf, zbuf16, msg_acc, den_acc, sem, sem2):
        cid = lax.axis_index("c")
        sid = lax.axis_index("s")
        wid = cid * NS + sid
        base_row = sid * rows_per_sub

        def zfill(i, carry):
            for j in range(HEADS):
                zbuf[i, pl.ds(j * 16, 16)] = jnp.zeros((16,), jnp.float32)
            zbuf16[i, :] = jnp.zeros((16,), jnp.float32)
            return carry
        lax.fori_loop(0, ZR, zfill, 0)

        def zcopy(t, carry):
            r = base_row + t * ZR
            pltpu.async_copy(zbuf, msg_acc.at[pl.ds(r, ZR)], sem)
            pltpu.async_copy(zbuf16, den_acc.at[pl.ds(r, ZR)], sem)
            return carry
        lax.fori_loop(0, rows_per_sub // ZR, zcopy, 0)

        def zdrain(t, carry):
            r = base_row + t * ZR
            pltpu.make_async_copy(zbuf, msg_acc.at[pl.ds(r, ZR)], sem).wait()
            pltpu.make_async_copy(zbuf16, den_acc.at[pl.ds(r, ZR)],
                                  sem).wait()
            return carry
        lax.fori_loop(0, rows_per_sub // ZR, zdrain, 0)
        plsc.subcore_barrier()

        row0 = wid * k_chunks

        def ring(k, carry):
            s = k % 2

            @pl.when((k >= 2) & (k <= k_chunks + 1))
            def _drain_scatter():
                pltpu.make_async_copy(
                    ex_rows, den_acc.at[dst_v.at[s]], sem2).wait()
                pltpu.make_async_copy(
                    xp_rows.at[pl.ds(s * CH, CH)],
                    msg_acc.at[dst_v.at[s]], sem2).wait()

            @pl.when(k < k_chunks)
            def _fire_gather():
                @pl.when(k % 12 == 0)
                def _refill_idx():
                    pltpu.sync_copy(pk_hbm.at[pl.ds(row0 + k, 12)], pk_v)
                for t in range(CH // 16):
                    v = pk_v[k % 12, pl.ds(t * 16, 16)]
                    src_v[s, pl.ds(t * 16, 16)] = v & 16383
                    dst_v[s, pl.ds(t * 16, 16)] = v >> 14
                pltpu.async_copy(xp_hbm.at[src_v.at[s]],
                                 xp_rows.at[pl.ds(s * CH, CH)], sem)
                pltpu.async_copy(sA_hbm.at[dst_v.at[s]],
                                 a_rows.at[pl.ds(s * CH, CH)], sem)
                pltpu.async_copy(sB_hbm.at[src_v.at[s]],
                                 b_rows.at[pl.ds(s * CH, CH)], sem)

            @pl.when((k >= 1) & (k <= k_chunks))
            def _compute():
                p = 1 - s
                pltpu.make_async_copy(xp_hbm.at[src_v.at[p]],
                                      xp_rows.at[pl.ds(p * CH, CH)],
                                      sem).wait()
                pltpu.make_async_copy(sA_hbm.at[dst_v.at[p]],
                                      a_rows.at[pl.ds(p * CH, CH)],
                                      sem).wait()
                pltpu.make_async_copy(sB_hbm.at[src_v.at[p]],
                                      b_rows.at[pl.ds(p * CH, CH)],
                                      sem).wait()
                base = p * CH

                @plsc.parallel_loop(0, CH, 1, unroll=4)
                def _edges(e):
                    i = base + e
                    v = a_rows[i, :] + b_rows[i, :]
                    v = jnp.where(v >= 0, v, NEG_SLOPE * v)
                    v = jnp.exp(v)
                    ex_rows[e, :] = v
                    for h in range(HEADS):
                        bro = v.at[jnp.full((16,), h, jnp.int32)].get(
                            mode="promise_in_bounds")
                        xp_rows[i, pl.ds(h * 16, 16)] = (
                            xp_rows[i, pl.ds(h * 16, 16)] * bro)

                pltpu.async_copy(ex_rows, den_acc.at[dst_v.at[p]], sem2,
                                 add=True)
                pltpu.async_copy(xp_rows.at[pl.ds(p * CH, CH)],
                                 msg_acc.at[dst_v.at[p]], sem2, add=True)
            return carry
        lax.fori_loop(0, k_chunks + 2, ring, 0)
        plsc.subcore_barrier()

        def wout(t, carry):
            r = base_row + t * ZR
            pltpu.async_copy(msg_acc.at[pl.ds(r, ZR)],
                             msg_out.at[cid, pl.ds(r, ZR)], sem)
            pltpu.async_copy(den_acc.at[pl.ds(r, ZR)],
                             den_out.at[cid, pl.ds(r, ZR)], sem)
            return carry
        lax.fori_loop(0, rows_per_sub // ZR, wout, 0)

        def wdrain(t, carry):
            r = base_row + t * ZR
            pltpu.make_async_copy(msg_acc.at[pl.ds(r, ZR)],
                                  msg_out.at[cid, pl.ds(r, ZR)], sem).wait()
            pltpu.make_async_copy(den_acc.at[pl.ds(r, ZR)],
                                  den_out.at[cid, pl.ds(r, ZR)], sem).wait()
            return carry
        lax.fori_loop(0, rows_per_sub // ZR, wdrain, 0)

    return body(xp, sA, sB, packed2d)


def kernel(x, edge_index, Wb1, A1, B1, att_src1, att_dst1, Asrc1, Bsrc1,
           Adst1, Bdst1, bias1, Wb2, A2, B2, att_src2, att_dst2, Asrc2,
           Bsrc2, Adst2, Bdst2, bias2):
    n = x.shape[0]
    e_raw = edge_index.shape[1]
    gran = NS * ZR
    npad = ((n + 1 + gran - 1) // gran) * gran
    blk = npad // 8
    assert npad <= 16384

    ep = e_raw + n
    per_round = NC * NS * CH
    k_chunks = (ep + per_round - 1) // per_round
    k_chunks = ((k_chunks + 11) // 12) * 12
    epad = k_chunks * per_round
    pad_e = epad - ep

    loops = jnp.arange(n, dtype=jnp.int32)
    src = jnp.concatenate([edge_index[0].astype(jnp.int32), loops,
                           jnp.zeros((pad_e,), jnp.int32)])
    dst = jnp.concatenate([edge_index[1].astype(jnp.int32), loops,
                           jnp.full((pad_e,), n, jnp.int32)])
    packed2d = (dst * 16384 + src).reshape(-1, CH)

    x_pad = jnp.pad(x, ((0, npad - n), (0, 0)))
    asrc1_f = att_src1.reshape(1, LANES)
    adst1_f = att_dst1.reshape(1, LANES)
    asrc2_f = att_src2.reshape(1, LANES)
    adst2_f = att_dst2.reshape(1, LANES)
    bias1_f = bias1.reshape(1, LANES)
    bias2_f = bias2.reshape(1, CDIM)

    xp1, sA1, sB1 = _tc_node1(x_pad, Wb1, A1, B1, asrc1_f, adst1_f,
                              Asrc1, Bsrc1, Adst1, Bdst1, npad, blk)
    msg1, den1 = _sc_edge(xp1, sA1, sB1, packed2d, npad, k_chunks)
    xp2, sA2, sB2 = _tc_mid(msg1, den1, x_pad, bias1_f, Wb2, A2, B2,
                            asrc2_f, adst2_f, Asrc2, Bsrc2, Adst2, Bdst2,
                            npad, blk)
    msg2, den2 = _sc_edge(xp2, sA2, sB2, packed2d, npad, k_chunks)
    out = _tc_final(msg2, den2, bias2_f, npad, blk)
    return out[:n]
